# Initial kernel scaffold; baseline (speedup 1.0000x reference)
import jax
import jax.numpy as jnp
from jax.experimental import pallas as pl


def kernel(x, router_w, router_b, gu_blocks, gu_scales, gu_bias, dn_blocks, dn_scales, dn_bias):
    raise NotImplementedError("write your pallas kernel here")


# speedup vs baseline: 3.5120x; 3.5120x over previous
import functools

import numpy as np
import jax
import jax.numpy as jnp
from jax import lax
from jax.experimental import pallas as pl
from jax.experimental.pallas import tpu as pltpu

ALPHA = 1.702
LIMIT = 7.0
FT = 512


def _nib2val(n):
    m = n & 7
    mag = jnp.where(
        m == 0, 0.0,
        jnp.where(m == 1, 0.5,
        jnp.where(m == 2, 1.0,
        jnp.where(m == 3, 1.5,
        jnp.where(m == 4, 2.0,
        jnp.where(m == 5, 3.0,
        jnp.where(m == 6, 4.0, 6.0)))))))
    return jnp.where(n >= 8, -mag, mag)


def _dot_nt(a, b):
    return lax.dot_general(a, b, (((1,), (1,)), ((), ())),
                           preferred_element_type=jnp.float32)


def _router_kernel(x_ref, rw_ref, rb_ref, wts_ref):
    x = x_ref[...]
    logits = _dot_nt(x, rw_ref[...]) + rb_ref[...]
    m1 = jnp.max(logits, axis=1, keepdims=True)
    is1 = logits == m1
    masked = jnp.where(is1, -jnp.inf, logits)
    m2 = jnp.max(masked, axis=1, keepdims=True)
    is2 = masked == m2
    p1 = 1.0 / (1.0 + jnp.exp(m2 - m1))
    wts_ref[...] = jnp.where(is1, p1, 0.0) + jnp.where(is2, 1.0 - p1, 0.0)


def _moe_kernel(xe_ref, xo_ref, wts_ref, gug_ref, guu_ref, gugs_ref,
                guus_ref, gb_ref, ub_ref, dnb_ref, dns_ref, dnbias_ref,
                out_ref):
    e = pl.program_id(0)
    j = pl.program_id(1)
    nsc = gugs_ref.shape[2]

    ci = lax.broadcasted_iota(jnp.int32, (nsc, 16 * nsc), 1) // 16
    bi = lax.broadcasted_iota(jnp.int32, (nsc, 16 * nsc), 0)
    sel16 = (ci == bi).astype(jnp.bfloat16)
    gsc = jnp.dot(gugs_ref[0].astype(jnp.bfloat16), sel16,
                  preferred_element_type=jnp.float32)
    usc = jnp.dot(guus_ref[0].astype(jnp.bfloat16), sel16,
                  preferred_element_type=jnp.float32)

    gbytes = gug_ref[0].astype(jnp.int32)
    ubytes = guu_ref[0].astype(jnp.int32)
    wg_lo = (_nib2val(gbytes & 15) * gsc).astype(jnp.bfloat16)
    wg_hi = (_nib2val(gbytes >> 4) * gsc).astype(jnp.bfloat16)
    wu_lo = (_nib2val(ubytes & 15) * usc).astype(jnp.bfloat16)
    wu_hi = (_nib2val(ubytes >> 4) * usc).astype(jnp.bfloat16)

    xe = xe_ref[...]
    xo = xo_ref[...]
    gate = _dot_nt(xe, wg_lo) + _dot_nt(xo, wg_hi) + gb_ref[0]
    up = _dot_nt(xe, wu_lo) + _dot_nt(xo, wu_hi) + ub_ref[0]

    gate = jnp.minimum(gate, LIMIT)
    up = jnp.clip(up, -LIMIT, LIMIT)
    glu = gate * (1.0 / (1.0 + jnp.exp(-ALPHA * gate)))
    act = ((up + 1.0) * glu).astype(jnp.bfloat16)

    db = lax.broadcasted_iota(jnp.int32, (dns_ref.shape[2], FT // 2), 0)
    dc = lax.broadcasted_iota(jnp.int32, (dns_ref.shape[2], FT // 2), 1) // 16
    selj = (db == j * (FT // 32) + dc).astype(jnp.bfloat16)
    dsc = jnp.dot(dns_ref[0].astype(jnp.bfloat16), selj,
                  preferred_element_type=jnp.float32)

    dbytes = dnb_ref[0].astype(jnp.int32)
    wd_lo = (_nib2val(dbytes & 15) * dsc).astype(jnp.bfloat16)
    wd_hi = (_nib2val(dbytes >> 4) * dsc).astype(jnp.bfloat16)

    half = FT // 2
    down = _dot_nt(act[:, :half], wd_lo) + _dot_nt(act[:, half:], wd_hi)

    wts = wts_ref[...]
    ei = lax.broadcasted_iota(jnp.int32, wts.shape, 1)
    w_col = jnp.sum(wts * (ei == e).astype(jnp.float32), axis=1,
                    keepdims=True)

    bias_gate = jnp.where(j == 0, 1.0, 0.0)
    contrib = w_col * (down + bias_gate * dnbias_ref[0])

    @pl.when(jnp.logical_and(e == 0, j == 0))
    def _():
        out_ref[...] = contrib

    @pl.when(jnp.logical_or(e != 0, j != 0))
    def _():
        out_ref[...] += contrib


@functools.partial(jax.jit, static_argnames=())
def kernel(x, router_w, router_b, gu_blocks, gu_scales, gu_bias, dn_blocks,
           dn_scales, dn_bias):
    Bb, Tt, H = x.shape
    E, FF2 = gu_bias.shape
    FF = FF2 // 2
    T = Bb * Tt
    J = FF // FT

    xf = x.reshape(T, H)
    xe = xf[:, 0::2].astype(jnp.bfloat16)
    xo = xf[:, 1::2].astype(jnp.bfloat16)

    idx = np.arange(FF)
    within = idx % FT
    base = (idx // FT) * FT
    P = base + np.where(within < FT // 2, 2 * within,
                        2 * (within - FT // 2) + 1)

    gu_b_flat = gu_blocks.reshape(E, 2 * FF, H // 2)
    gug = gu_b_flat[:, 2 * P, :]
    guu = gu_b_flat[:, 2 * P + 1, :]
    gu_s = jnp.exp2(gu_scales.astype(jnp.float32) - 127.0)
    gugs = gu_s[:, 2 * P, :]
    guus = gu_s[:, 2 * P + 1, :]
    gb = gu_bias[:, 2 * P].reshape(E, 1, FF)
    ub = gu_bias[:, 2 * P + 1].reshape(E, 1, FF)

    dnb = dn_blocks.reshape(E, H, FF // 2)
    dns = jnp.exp2(dn_scales.astype(jnp.float32) - 127.0)
    dnbias = dn_bias.reshape(E, 1, H)

    wts = pl.pallas_call(
        _router_kernel,
        out_shape=jax.ShapeDtypeStruct((T, E), jnp.float32),
    )(xf, router_w, router_b.reshape(1, E))

    out = pl.pallas_call(
        _moe_kernel,
        grid=(E, J),
        in_specs=[
            pl.BlockSpec((T, H // 2), lambda e, j: (0, 0)),
            pl.BlockSpec((T, H // 2), lambda e, j: (0, 0)),
            pl.BlockSpec((T, E), lambda e, j: (0, 0)),
            pl.BlockSpec((1, FT, H // 2), lambda e, j: (e, j, 0)),
            pl.BlockSpec((1, FT, H // 2), lambda e, j: (e, j, 0)),
            pl.BlockSpec((1, FT, H // 32), lambda e, j: (e, j, 0)),
            pl.BlockSpec((1, FT, H // 32), lambda e, j: (e, j, 0)),
            pl.BlockSpec((1, 1, FT), lambda e, j: (e, 0, j)),
            pl.BlockSpec((1, 1, FT), lambda e, j: (e, 0, j)),
            pl.BlockSpec((1, H, FT // 2), lambda e, j: (e, 0, j)),
            pl.BlockSpec((1, H, FF // 32), lambda e, j: (e, 0, 0)),
            pl.BlockSpec((1, 1, H), lambda e, j: (e, 0, 0)),
        ],
        out_specs=pl.BlockSpec((T, H), lambda e, j: (0, 0)),
        out_shape=jax.ShapeDtypeStruct((T, H), jnp.float32),
    )(xe, xo, wts, gug, guu, gugs, guus, gb, ub, dnb, dns, dnbias)

    return out.reshape(Bb, Tt, H)
